# Initial kernel scaffold; baseline (speedup 1.0000x reference)
import jax
import jax.numpy as jnp
from jax.experimental import pallas as pl


def kernel(feature_0, feature_1, feature_2, feature_3, feature_4, feature_5, feature_6, feature_7, feature_8, feature_9, feature_10, feature_11, feature_12, feature_13, feature_14, feature_15, feature_16, feature_17, feature_18, feature_19, feature_20, feature_21, feature_22, feature_23, feature_24, feature_25, table_0, table_1, table_2, table_3, table_4, table_5, table_6, table_7, table_8, table_9, table_10, table_11, table_12, table_13, table_14, table_15, table_16, table_17, table_18, table_19, table_20, table_21, table_22, table_23, table_24, table_25):
    raise NotImplementedError("write your pallas kernel here")


# speedup vs baseline: 31.3363x; 31.3363x over previous
import functools

import jax
import jax.numpy as jnp
from jax import lax
from jax.experimental import pallas as pl
from jax.experimental.pallas import tpu as pltpu
from jax.experimental.pallas import tpu_sc as plsc

N_FEAT = 26
BATCH = 16384
VOCAB = 10
DIM = 3

NC = 2
NS = 16
NW = NC * NS
BPW = BATCH // NW
L = 16
NVEC = BPW // L

_mesh = plsc.VectorSubcoreMesh(core_axis_name="c", subcore_axis_name="s")


@functools.partial(
    pl.kernel,
    out_type=jax.ShapeDtypeStruct((BATCH, DIM), jnp.float32),
    mesh=_mesh,
    compiler_params=pltpu.CompilerParams(needs_layout_passes=False),
    scratch_types=[
        pltpu.VMEM((N_FEAT, BPW), jnp.int32),
        pltpu.VMEM((N_FEAT * VOCAB * DIM,), jnp.float32),
        pltpu.VMEM((BPW, DIM), jnp.float32),
        pltpu.SemaphoreType.DMA,
    ],
)
def _merger(*refs):
    feats = refs[:N_FEAT]
    tab_hbm = refs[N_FEAT]
    out_hbm = refs[N_FEAT + 1]
    feat_v, tab_v, out_v, sem = refs[N_FEAT + 2:]

    wid = lax.axis_index("s") * NC + lax.axis_index("c")
    base = wid * BPW

    copies = []
    for i in range(N_FEAT):
        copies.append(
            pltpu.make_async_copy(feats[i].at[pl.ds(base, BPW)], feat_v.at[i], sem)
        )
    copies.append(pltpu.make_async_copy(tab_hbm, tab_v, sem))
    for c in copies:
        c.start()
    for c in copies:
        c.wait()

    def body(j, carry):
        col = j * L
        acc = [jnp.zeros((L,), jnp.float32) for _ in range(DIM)]
        for i in range(N_FEAT):
            f3 = feat_v[i, pl.ds(col, L)] * 3
            for d in range(DIM):
                idx = f3 + (i * VOCAB * DIM + d)
                acc[d] = acc[d] + plsc.load_gather(tab_v, [idx])
        rows = col + lax.iota(jnp.int32, L)
        for d in range(DIM):
            plsc.store_scatter(out_v, [rows, jnp.full((L,), d, jnp.int32)], acc[d])
        return carry

    lax.fori_loop(0, NVEC, body, 0)
    pltpu.sync_copy(out_v, out_hbm.at[pl.ds(base, BPW)])


def kernel(*args):
    feats = args[:N_FEAT]
    tabs = args[N_FEAT:2 * N_FEAT]
    tab_flat = jnp.stack(tabs).reshape(-1)
    return _merger(*feats, tab_flat)
